# Initial kernel scaffold; baseline (speedup 1.0000x reference)
import jax
import jax.numpy as jnp
from jax import lax
from jax.experimental import pallas as pl
from jax.experimental.pallas import tpu as pltpu
from jax.experimental.pallas import tpu_sc as plsc

C = 128
K = 15
CPG = 16
GROUPS = 8
H = 32
SIGMA = 2.0
N = 10000
M = 10000

NC = 2
NS = 16
NW = NC * NS
QB = 4
NCHUNK = M // QB
MODW = 256


def _mlp_body(x_ref, w1_ref, b1_ref, w2_ref, o_ref):
    x = x_ref[...]
    h = jnp.dot(x, w1_ref[...], preferred_element_type=jnp.float32)
    h = h + b1_ref[...]
    h = jnp.where(h > 0, h, h * 0.1)
    z = jnp.dot(h, w2_ref[...], preferred_element_type=jnp.float32)
    o_ref[...] = 1.0 / (1.0 + jnp.exp(-z))


def _modulations(s_feats, W1, b1, W2p):
    mb = 1000
    return pl.pallas_call(
        _mlp_body,
        grid=(M // mb,),
        in_specs=[
            pl.BlockSpec((mb, C), lambda i: (i, 0)),
            pl.BlockSpec((C, C), lambda i: (0, 0)),
            pl.BlockSpec((1, C), lambda i: (0, 0)),
            pl.BlockSpec((C, MODW), lambda i: (0, 0)),
        ],
        out_specs=pl.BlockSpec((mb, MODW), lambda i: (i, 0)),
        out_shape=jax.ShapeDtypeStruct((M, MODW), jnp.float32),
    )(s_feats, W1, b1.reshape(1, C), W2p)


def _sc_body(nb_hbm, qp_hbm, spts_hbm, sfeats_hbm, wts_hbm, kp_hbm, mod_hbm,
             out_hbm,
             spts_v, wts_v, kp_v, nb_v, qp_v, mod_v, feat_v, k_v, i_v, out_v,
             sem):
    wid = lax.axis_index("c") * NS + lax.axis_index("s")

    pltpu.sync_copy(spts_hbm, spts_v)
    pltpu.sync_copy(wts_hbm, wts_v)
    pltpu.sync_copy(kp_hbm, kp_v)

    nchunks = (NCHUNK - wid + NW - 1) // NW

    def chunk_body(j, carry):
        c = wid + NW * j
        base = c * QB
        pltpu.sync_copy(nb_hbm.at[pl.ds(base * H, QB * H)], nb_v)
        pltpu.sync_copy(qp_hbm.at[pl.ds(base, QB)], qp_v)
        pltpu.sync_copy(mod_hbm.at[pl.ds(base * MODW, QB * MODW)], mod_v)
        pltpu.async_copy(sfeats_hbm.at[nb_v], feat_v, sem).wait()

        def q_body(i, carry_q):
            qx = qp_v[i, 0]
            qy = qp_v[i, 1]
            qz = qp_v[i, 2]

            def geo_half(jh, carry_g):
                t16 = i * H + jh * 16
                idx16 = nb_v[pl.ds(t16, 16)]
                xs = plsc.load_gather(spts_v, [idx16])
                ys = plsc.load_gather(spts_v, [idx16 + N])
                zs = plsc.load_gather(spts_v, [idx16 + 2 * N])
                dx = xs - qx
                dy = ys - qy
                dz = zs - qz
                best = jnp.full((16,), 1e30, jnp.float32)
                bestk = jnp.zeros((16,), jnp.int32)
                for k in range(K):
                    ddx = dx - kp_v[k, 0]
                    ddy = dy - kp_v[k, 1]
                    ddz = dz - kp_v[k, 2]
                    d2 = ddx * ddx + ddy * ddy + ddz * ddz
                    m = d2 < best
                    best = jnp.where(m, d2, best)
                    bestk = jnp.where(m, jnp.int32(k), bestk)
                x = jnp.maximum(best, jnp.float32(1e-24))
                xi = plsc.bitcast(x, jnp.int32)
                r = plsc.bitcast(jnp.int32(0x5F3759DF) - (xi >> 1),
                                 jnp.float32)
                for _ in range(3):
                    r = r * (1.5 - 0.5 * x * r * r)
                s = x * r
                infl = jnp.maximum(1.0 - s * jnp.float32(1.0 / SIGMA), 0.0)
                k_v[pl.ds(t16, 16)] = bestk * CPG
                i_v[pl.ds(t16, 16)] = infl
                return carry_g

            lax.fori_loop(0, 2, geo_half, 0)

            def h_body(h, accs):
                n = i * H + h
                kofs = k_v[n]
                fl = i_v[n]
                m16 = mod_v[pl.ds(i * MODW + kofs, 16)]
                modi = m16 * fl
                out = []
                for g in range(GROUPS):
                    w = wts_v[pl.ds(kofs * GROUPS + g * CPG, 16)]
                    f = feat_v[n, pl.ds(g * CPG, 16)]
                    out.append(accs[g] + f * w * modi)
                return tuple(out)

            accs = lax.fori_loop(
                0, H, h_body,
                tuple(jnp.zeros((16,), jnp.float32) for _ in range(GROUPS)))
            for g in range(GROUPS):
                out_v[i, pl.ds(g * CPG, 16)] = accs[g]
            return carry_q

        lax.fori_loop(0, QB, q_body, 0)
        pltpu.sync_copy(out_v, out_hbm.at[pl.ds(base, QB)])
        return carry

    lax.fori_loop(0, nchunks, chunk_body, 0)


def _sc_main(nb_flat, qp_pad, spts_flat, s_feats, wts_flat, kp, mod_flat):
    mesh = plsc.VectorSubcoreMesh(core_axis_name="c", subcore_axis_name="s",
                                  num_cores=NC, num_subcores=NS)
    return pl.kernel(
        _sc_body,
        out_type=jax.ShapeDtypeStruct((M, C), jnp.float32),
        mesh=mesh,
        scratch_types=[
            pltpu.VMEM((3 * N,), jnp.float32),
            pltpu.VMEM((K * C,), jnp.float32),
            pltpu.VMEM((K, 3), jnp.float32),
            pltpu.VMEM((QB * H,), jnp.int32),
            pltpu.VMEM((QB, 4), jnp.float32),
            pltpu.VMEM((QB * MODW,), jnp.float32),
            pltpu.VMEM((QB * H, C), jnp.float32),
            pltpu.VMEM((QB * H,), jnp.int32),
            pltpu.VMEM((QB * H,), jnp.float32),
            pltpu.VMEM((QB, C), jnp.float32),
            pltpu.SemaphoreType.DMA,
        ],
    )(nb_flat, qp_pad, spts_flat, s_feats, wts_flat, kp, mod_flat)


def kernel(q_pts, s_pts, s_feats, neighb_inds, weights, W1, b1, W2,
           kernel_points):
    nb_flat = neighb_inds.astype(jnp.int32).reshape(-1)
    qp_pad = jnp.pad(q_pts, ((0, 0), (0, 1)))
    spts_flat = s_pts.T.reshape(-1)
    wts_flat = weights.reshape(-1)
    W2p = jnp.pad(W2, ((0, 0), (0, MODW - K * CPG)))
    mod = _modulations(s_feats, W1, b1, W2p)
    return _sc_main(nb_flat, qp_pad, spts_flat, s_feats, wts_flat,
                    kernel_points, mod.reshape(-1))


# speedup vs baseline: 2.9331x; 2.9331x over previous
import jax
import jax.numpy as jnp
from jax import lax
from jax.experimental import pallas as pl
from jax.experimental.pallas import tpu as pltpu
from jax.experimental.pallas import tpu_sc as plsc

C = 128
K = 15
CPG = 16
GROUPS = 8
H = 32
SIGMA = 2.0
N = 10000
M = 10000

NC = 2
NS = 16
NW = NC * NS
QB = 4
NCHUNK = M // QB
MODW = 256


def _mlp_body(x_ref, w1_ref, b1_ref, w2_ref, o_ref):
    x = x_ref[...]
    h = jnp.dot(x, w1_ref[...], preferred_element_type=jnp.float32)
    h = h + b1_ref[...]
    h = jnp.where(h > 0, h, h * 0.1)
    z = jnp.dot(h, w2_ref[...], preferred_element_type=jnp.float32)
    o_ref[...] = 1.0 / (1.0 + jnp.exp(-z))


def _modulations(s_feats, W1, b1, W2p):
    mb = 1000
    return pl.pallas_call(
        _mlp_body,
        grid=(M // mb,),
        in_specs=[
            pl.BlockSpec((mb, C), lambda i: (i, 0)),
            pl.BlockSpec((C, C), lambda i: (0, 0)),
            pl.BlockSpec((1, C), lambda i: (0, 0)),
            pl.BlockSpec((C, MODW), lambda i: (0, 0)),
        ],
        out_specs=pl.BlockSpec((mb, MODW), lambda i: (i, 0)),
        out_shape=jax.ShapeDtypeStruct((M, MODW), jnp.float32),
    )(s_feats, W1, b1.reshape(1, C), W2p)


def _take_splat(vec, idx):
    return vec.at[jnp.full((16,), 0, jnp.int32) + idx].get(
        mode="promise_in_bounds")


def _sc_body(nb_hbm, qp_hbm, spts_hbm, sfeats_hbm, wts_hbm, kp_hbm, mod_hbm,
             out_hbm,
             spts_v, wts_v, kp_v, nb_v, qp_v, mod_v, feat_v, out_v,
             sem):
    wid = lax.axis_index("c") * NS + lax.axis_index("s")

    pltpu.sync_copy(spts_hbm, spts_v)
    pltpu.sync_copy(wts_hbm, wts_v)
    pltpu.sync_copy(kp_hbm, kp_v)

    kpxv = kp_v[pl.ds(0, 16)]
    kpyv = kp_v[pl.ds(16, 16)]
    kpzv = kp_v[pl.ds(32, 16)]
    kpx = [kpxv[k] for k in range(K)]
    kpy = [kpyv[k] for k in range(K)]
    kpz = [kpzv[k] for k in range(K)]

    nchunks = (NCHUNK - wid + NW - 1) // NW

    def chunk_body(j, carry):
        c = wid + NW * j
        base = c * QB
        pltpu.sync_copy(nb_hbm.at[pl.ds(base * H, QB * H)], nb_v)
        pltpu.sync_copy(qp_hbm.at[pl.ds(base * 4, QB * 4)], qp_v)
        pltpu.sync_copy(mod_hbm.at[pl.ds(base * MODW, QB * MODW)], mod_v)
        pltpu.async_copy(sfeats_hbm.at[nb_v], feat_v, sem).wait()
        qall = qp_v[pl.ds(0, 16)]

        def q_body(i, carry_q):
            qxs = _take_splat(qall, i * 4)
            qys = _take_splat(qall, i * 4 + 1)
            qzs = _take_splat(qall, i * 4 + 2)

            def half_body(jh, accs):
                t16 = i * H + jh * 16
                idx16 = nb_v[pl.ds(t16, 16)]
                xs = plsc.load_gather(spts_v, [idx16])
                ys = plsc.load_gather(spts_v, [idx16 + N])
                zs = plsc.load_gather(spts_v, [idx16 + 2 * N])
                dx = xs - qxs
                dy = ys - qys
                dz = zs - qzs
                best = jnp.full((16,), 1e30, jnp.float32)
                bestk = jnp.zeros((16,), jnp.int32)
                for k in range(K):
                    ddx = dx - kpx[k]
                    ddy = dy - kpy[k]
                    ddz = dz - kpz[k]
                    d2 = ddx * ddx + ddy * ddy + ddz * ddz
                    m = d2 < best
                    best = jnp.where(m, d2, best)
                    bestk = jnp.where(m, jnp.int32(k), bestk)
                x = jnp.maximum(best, jnp.float32(1e-24))
                xi = plsc.bitcast(x, jnp.int32)
                r = plsc.bitcast(jnp.int32(0x5F3759DF) - (xi >> 1),
                                 jnp.float32)
                for _ in range(3):
                    r = r * (1.5 - 0.5 * x * r * r)
                s = x * r
                infl16 = jnp.maximum(1.0 - s * jnp.float32(1.0 / SIGMA), 0.0)
                kofs16 = bestk * CPG

                accs = list(accs)
                for l in range(16):
                    kofs = kofs16[l]
                    fl = infl16[l]
                    m16 = mod_v[pl.ds(i * MODW + kofs, 16)]
                    modi = m16 * fl
                    n = t16 + l
                    for g in range(GROUPS):
                        w = wts_v[pl.ds(kofs * GROUPS + g * CPG, 16)]
                        f = feat_v[n, pl.ds(g * CPG, 16)]
                        accs[g] = accs[g] + f * w * modi
                return tuple(accs)

            accs = lax.fori_loop(
                0, 2, half_body,
                tuple(jnp.zeros((16,), jnp.float32) for _ in range(GROUPS)))
            for g in range(GROUPS):
                out_v[i, pl.ds(g * CPG, 16)] = accs[g]
            return carry_q

        lax.fori_loop(0, QB, q_body, 0)
        pltpu.sync_copy(out_v, out_hbm.at[pl.ds(base, QB)])
        return carry

    lax.fori_loop(0, nchunks, chunk_body, 0)


def _sc_main(nb_flat, qp_pad, spts_flat, s_feats, wts_flat, kp, mod_flat):
    mesh = plsc.VectorSubcoreMesh(core_axis_name="c", subcore_axis_name="s",
                                  num_cores=NC, num_subcores=NS)
    return pl.kernel(
        _sc_body,
        out_type=jax.ShapeDtypeStruct((M, C), jnp.float32),
        mesh=mesh,
        compiler_params=pltpu.CompilerParams(needs_layout_passes=False),
        scratch_types=[
            pltpu.VMEM((3 * N,), jnp.float32),
            pltpu.VMEM((K * C,), jnp.float32),
            pltpu.VMEM((48,), jnp.float32),
            pltpu.VMEM((QB * H,), jnp.int32),
            pltpu.VMEM((QB * 4,), jnp.float32),
            pltpu.VMEM((QB * MODW,), jnp.float32),
            pltpu.VMEM((QB * H, C), jnp.float32),
            pltpu.VMEM((QB, C), jnp.float32),
            pltpu.SemaphoreType.DMA,
        ],
    )(nb_flat, qp_pad, spts_flat, s_feats, wts_flat, kp, mod_flat)


def kernel(q_pts, s_pts, s_feats, neighb_inds, weights, W1, b1, W2,
           kernel_points):
    nb_flat = neighb_inds.astype(jnp.int32).reshape(-1)
    qp_flat = jnp.pad(q_pts, ((0, 0), (0, 1))).reshape(-1)
    spts_flat = s_pts.T.reshape(-1)
    wts_flat = weights.reshape(-1)
    kp_flat = jnp.pad(kernel_points, ((0, 16 - K), (0, 0))).T.reshape(-1)
    W2p = jnp.pad(W2, ((0, 0), (0, MODW - K * CPG)))
    mod = _modulations(s_feats, W1, b1, W2p)
    return _sc_main(nb_flat, qp_flat, spts_flat, s_feats, wts_flat,
                    kp_flat, mod.reshape(-1))


# speedup vs baseline: 3.7982x; 1.2949x over previous
import jax
import jax.numpy as jnp
from jax import lax
from jax.experimental import pallas as pl
from jax.experimental.pallas import tpu as pltpu
from jax.experimental.pallas import tpu_sc as plsc

C = 128
K = 15
CPG = 16
GROUPS = 8
H = 32
SIGMA = 2.0
N = 10000
M = 10000

NC = 2
NS = 16
NW = NC * NS
QB = 4
NCHUNK = M // QB
MODW = 256


def _mlp_body(x_ref, w1_ref, b1_ref, w2_ref, o_ref):
    x = x_ref[...]
    h = jnp.dot(x, w1_ref[...], preferred_element_type=jnp.float32)
    h = h + b1_ref[...]
    h = jnp.where(h > 0, h, h * 0.1)
    z = jnp.dot(h, w2_ref[...], preferred_element_type=jnp.float32)
    o_ref[...] = 1.0 / (1.0 + jnp.exp(-z))


def _modulations(s_feats, W1, b1, W2p):
    mb = 1000
    return pl.pallas_call(
        _mlp_body,
        grid=(M // mb,),
        in_specs=[
            pl.BlockSpec((mb, C), lambda i: (i, 0)),
            pl.BlockSpec((C, C), lambda i: (0, 0)),
            pl.BlockSpec((1, C), lambda i: (0, 0)),
            pl.BlockSpec((C, MODW), lambda i: (0, 0)),
        ],
        out_specs=pl.BlockSpec((mb, MODW), lambda i: (i, 0)),
        out_shape=jax.ShapeDtypeStruct((M, MODW), jnp.float32),
    )(s_feats, W1, b1.reshape(1, C), W2p)


def _take_splat(vec, idx):
    return vec.at[jnp.full((16,), 0, jnp.int32) + idx].get(
        mode="promise_in_bounds")


def _sc_body(nb_hbm, qp_hbm, spts_hbm, sfeats_hbm, wts_hbm, kp_hbm, mod_hbm,
             out_hbm,
             spts_v, wts_v, kp_v, nb_v, qp_v, mod_v, feat_v, out_v,
             sem_i, sem_f, sem_o):
    wid = lax.axis_index("c") * NS + lax.axis_index("s")

    pltpu.sync_copy(spts_hbm, spts_v)
    pltpu.sync_copy(wts_hbm, wts_v)
    pltpu.sync_copy(kp_hbm, kp_v)

    kpxv = kp_v[pl.ds(0, 16)]
    kpyv = kp_v[pl.ds(16, 16)]
    kpzv = kp_v[pl.ds(32, 16)]
    kpx = [kpxv[k] for k in range(K)]
    kpy = [kpyv[k] for k in range(K)]
    kpz = [kpzv[k] for k in range(K)]

    nchunks = (NCHUNK - wid + NW - 1) // NW

    def issue_idx(c):
        s = c % 3
        base = (wid + NW * c) * QB
        pltpu.async_copy(nb_hbm.at[pl.ds(base * H, QB * H)], nb_v.at[s],
                         sem_i)
        pltpu.async_copy(qp_hbm.at[pl.ds(base * 4, QB * 4)], qp_v.at[s],
                         sem_i)
        pltpu.async_copy(mod_hbm.at[pl.ds(base * MODW, QB * MODW)],
                         mod_v.at[s], sem_i)

    def wait_idx(c):
        s = c % 3
        pltpu.make_async_copy(nb_hbm.at[pl.ds(0, QB * H)], nb_v.at[s],
                              sem_i).wait()
        pltpu.make_async_copy(qp_hbm.at[pl.ds(0, QB * 4)], qp_v.at[s],
                              sem_i).wait()
        pltpu.make_async_copy(mod_hbm.at[pl.ds(0, QB * MODW)], mod_v.at[s],
                              sem_i).wait()

    def issue_feat(c):
        pltpu.async_copy(sfeats_hbm.at[nb_v.at[c % 3]], feat_v.at[c % 2],
                         sem_f)

    def wait_feat(c):
        pltpu.make_async_copy(sfeats_hbm.at[nb_v.at[c % 3]],
                              feat_v.at[c % 2], sem_f).wait()

    def wait_out(c):
        base = (wid + NW * c) * QB
        pltpu.make_async_copy(out_v, out_hbm.at[pl.ds(base, QB)],
                              sem_o).wait()

    issue_idx(0)

    @pl.when(nchunks > 1)
    def _():
        issue_idx(1)

    wait_idx(0)
    issue_feat(0)

    def chunk_body(c, carry):
        sf = c % 2
        si = c % 3
        base = (wid + NW * c) * QB
        wait_feat(c)

        @pl.when(c + 1 < nchunks)
        def _():
            wait_idx(c + 1)

        @pl.when(c + 2 < nchunks)
        def _():
            issue_idx(c + 2)

        @pl.when(c + 1 < nchunks)
        def _():
            issue_feat(c + 1)

        @pl.when(c > 0)
        def _():
            wait_out(c - 1)

        qall = qp_v[si, pl.ds(0, 16)]

        def q_body(i, carry_q):
            qxs = _take_splat(qall, i * 4)
            qys = _take_splat(qall, i * 4 + 1)
            qzs = _take_splat(qall, i * 4 + 2)

            def half_body(jh, accs):
                t16 = i * H + jh * 16
                idx16 = nb_v[si, pl.ds(t16, 16)]
                xs = plsc.load_gather(spts_v, [idx16])
                ys = plsc.load_gather(spts_v, [idx16 + N])
                zs = plsc.load_gather(spts_v, [idx16 + 2 * N])
                dx = xs - qxs
                dy = ys - qys
                dz = zs - qzs
                best = jnp.full((16,), 1e30, jnp.float32)
                bestk = jnp.zeros((16,), jnp.int32)
                for k in range(K):
                    ddx = dx - kpx[k]
                    ddy = dy - kpy[k]
                    ddz = dz - kpz[k]
                    d2 = ddx * ddx + ddy * ddy + ddz * ddz
                    m = d2 < best
                    best = jnp.where(m, d2, best)
                    bestk = jnp.where(m, jnp.int32(k), bestk)
                x = jnp.maximum(best, jnp.float32(1e-24))
                xi = plsc.bitcast(x, jnp.int32)
                r = plsc.bitcast(jnp.int32(0x5F3759DF) - (xi >> 1),
                                 jnp.float32)
                for _ in range(3):
                    r = r * (1.5 - 0.5 * x * r * r)
                s = x * r
                infl16 = jnp.maximum(1.0 - s * jnp.float32(1.0 / SIGMA), 0.0)
                kofs16 = bestk * CPG

                accs = list(accs)
                for l in range(16):
                    kofs = kofs16[l]
                    fl = infl16[l]
                    m16 = mod_v[si, pl.ds(i * MODW + kofs, 16)]
                    modi = m16 * fl
                    n = t16 + l
                    for g in range(GROUPS):
                        w = wts_v[pl.ds(kofs * GROUPS + g * CPG, 16)]
                        f = feat_v[sf, n, pl.ds(g * CPG, 16)]
                        accs[g] = accs[g] + f * w * modi
                return tuple(accs)

            accs = lax.fori_loop(
                0, 2, half_body,
                tuple(jnp.zeros((16,), jnp.float32) for _ in range(GROUPS)))
            for g in range(GROUPS):
                out_v[i, pl.ds(g * CPG, 16)] = accs[g]
            return carry_q

        lax.fori_loop(0, QB, q_body, 0)
        pltpu.async_copy(out_v, out_hbm.at[pl.ds(base, QB)], sem_o)
        return carry

    lax.fori_loop(0, nchunks, chunk_body, 0)
    wait_out(nchunks - 1)


def _sc_main(nb_flat, qp_pad, spts_flat, s_feats, wts_flat, kp, mod_flat):
    mesh = plsc.VectorSubcoreMesh(core_axis_name="c", subcore_axis_name="s",
                                  num_cores=NC, num_subcores=NS)
    return pl.kernel(
        _sc_body,
        out_type=jax.ShapeDtypeStruct((M, C), jnp.float32),
        mesh=mesh,
        compiler_params=pltpu.CompilerParams(needs_layout_passes=False),
        scratch_types=[
            pltpu.VMEM((3 * N,), jnp.float32),
            pltpu.VMEM((K * C,), jnp.float32),
            pltpu.VMEM((48,), jnp.float32),
            pltpu.VMEM((3, QB * H), jnp.int32),
            pltpu.VMEM((3, QB * 4), jnp.float32),
            pltpu.VMEM((3, QB * MODW), jnp.float32),
            pltpu.VMEM((2, QB * H, C), jnp.float32),
            pltpu.VMEM((QB, C), jnp.float32),
            pltpu.SemaphoreType.DMA,
            pltpu.SemaphoreType.DMA,
            pltpu.SemaphoreType.DMA,
        ],
    )(nb_flat, qp_pad, spts_flat, s_feats, wts_flat, kp, mod_flat)


def kernel(q_pts, s_pts, s_feats, neighb_inds, weights, W1, b1, W2,
           kernel_points):
    nb_flat = neighb_inds.astype(jnp.int32).reshape(-1)
    qp_flat = jnp.pad(q_pts, ((0, 0), (0, 1))).reshape(-1)
    spts_flat = s_pts.T.reshape(-1)
    wts_flat = weights.reshape(-1)
    kp_flat = jnp.pad(kernel_points, ((0, 16 - K), (0, 0))).T.reshape(-1)
    W2p = jnp.pad(W2, ((0, 0), (0, MODW - K * CPG)))
    mod = _modulations(s_feats, W1, b1, W2p)
    return _sc_main(nb_flat, qp_flat, spts_flat, s_feats, wts_flat,
                    kp_flat, mod.reshape(-1))


# speedup vs baseline: 4.0230x; 1.0592x over previous
import jax
import jax.numpy as jnp
from jax import lax
from jax.experimental import pallas as pl
from jax.experimental.pallas import tpu as pltpu
from jax.experimental.pallas import tpu_sc as plsc

C = 128
K = 15
CPG = 16
GROUPS = 8
H = 32
SIGMA = 2.0
N = 10000
M = 10000

NC = 2
NS = 16
NW = NC * NS
QB = 8
NCHUNK = M // QB
MODW = 256


def _mlp_body(x_ref, w1_ref, b1_ref, w2_ref, o_ref):
    x = x_ref[...]
    h = jnp.dot(x, w1_ref[...], preferred_element_type=jnp.float32)
    h = h + b1_ref[...]
    h = jnp.where(h > 0, h, h * 0.1)
    z = jnp.dot(h, w2_ref[...], preferred_element_type=jnp.float32)
    o_ref[...] = 1.0 / (1.0 + jnp.exp(-z))


def _modulations(s_feats, W1, b1, W2p):
    mb = 1000
    return pl.pallas_call(
        _mlp_body,
        grid=(M // mb,),
        in_specs=[
            pl.BlockSpec((mb, C), lambda i: (i, 0)),
            pl.BlockSpec((C, C), lambda i: (0, 0)),
            pl.BlockSpec((1, C), lambda i: (0, 0)),
            pl.BlockSpec((C, MODW), lambda i: (0, 0)),
        ],
        out_specs=pl.BlockSpec((mb, MODW), lambda i: (i, 0)),
        out_shape=jax.ShapeDtypeStruct((M, MODW), jnp.float32),
    )(s_feats, W1, b1.reshape(1, C), W2p)


def _take_splat(vec, idx):
    return vec.at[jnp.full((16,), 0, jnp.int32) + idx].get(
        mode="promise_in_bounds")


def _sc_body(nb_hbm, qp_hbm, spts_hbm, sfeats_hbm, wts_hbm, kp_hbm, mod_hbm,
             out_hbm,
             spts_v, wts_v, kp_v, nb_v, qp_v, mod_v, feat_v, out_v,
             sem_i, sem_f, sem_o):
    wid = lax.axis_index("c") * NS + lax.axis_index("s")

    pltpu.sync_copy(spts_hbm, spts_v)
    pltpu.sync_copy(wts_hbm, wts_v)
    pltpu.sync_copy(kp_hbm, kp_v)

    kpxv = kp_v[pl.ds(0, 16)]
    kpyv = kp_v[pl.ds(16, 16)]
    kpzv = kp_v[pl.ds(32, 16)]
    kpx = [kpxv[k] for k in range(K)]
    kpy = [kpyv[k] for k in range(K)]
    kpz = [kpzv[k] for k in range(K)]

    nchunks = (NCHUNK - wid + NW - 1) // NW

    def issue_idx(c):
        s = c % 3
        base = (wid + NW * c) * QB
        pltpu.async_copy(nb_hbm.at[pl.ds(base * H, QB * H)], nb_v.at[s],
                         sem_i)
        pltpu.async_copy(qp_hbm.at[pl.ds(base * 4, QB * 4)], qp_v.at[s],
                         sem_i)
        pltpu.async_copy(mod_hbm.at[pl.ds(base * MODW, QB * MODW)],
                         mod_v.at[s], sem_i)

    def wait_idx(c):
        s = c % 3
        pltpu.make_async_copy(nb_hbm.at[pl.ds(0, QB * H)], nb_v.at[s],
                              sem_i).wait()
        pltpu.make_async_copy(qp_hbm.at[pl.ds(0, QB * 4)], qp_v.at[s],
                              sem_i).wait()
        pltpu.make_async_copy(mod_hbm.at[pl.ds(0, QB * MODW)], mod_v.at[s],
                              sem_i).wait()

    def issue_feat(c):
        s, sf = c % 3, c % 2
        pltpu.async_copy(sfeats_hbm.at[nb_v.at[s, pl.ds(0, 128)]],
                         feat_v.at[sf, pl.ds(0, 128)], sem_f)
        pltpu.async_copy(sfeats_hbm.at[nb_v.at[s, pl.ds(128, 128)]],
                         feat_v.at[sf, pl.ds(128, 128)], sem_f)

    def wait_feat(c):
        s, sf = c % 3, c % 2
        pltpu.make_async_copy(sfeats_hbm.at[nb_v.at[s, pl.ds(0, 128)]],
                              feat_v.at[sf, pl.ds(0, 128)], sem_f).wait()
        pltpu.make_async_copy(sfeats_hbm.at[nb_v.at[s, pl.ds(128, 128)]],
                              feat_v.at[sf, pl.ds(128, 128)], sem_f).wait()

    def wait_out(c):
        base = (wid + NW * c) * QB
        pltpu.make_async_copy(out_v, out_hbm.at[pl.ds(base, QB)],
                              sem_o).wait()

    issue_idx(0)

    @pl.when(nchunks > 1)
    def _():
        issue_idx(1)

    wait_idx(0)
    issue_feat(0)

    def chunk_body(c, carry):
        sf = c % 2
        si = c % 3
        base = (wid + NW * c) * QB
        wait_feat(c)

        @pl.when(c + 1 < nchunks)
        def _():
            wait_idx(c + 1)

        @pl.when(c + 2 < nchunks)
        def _():
            issue_idx(c + 2)

        @pl.when(c + 1 < nchunks)
        def _():
            issue_feat(c + 1)

        @pl.when(c > 0)
        def _():
            wait_out(c - 1)

        qall0 = qp_v[si, pl.ds(0, 16)]
        qall1 = qp_v[si, pl.ds(16, 16)]

        def q_body(i, carry_q):
            qv = jnp.where(i < 4, qall0, qall1)
            qo = (i % 4) * 4
            qxs = _take_splat(qv, qo)
            qys = _take_splat(qv, qo + 1)
            qzs = _take_splat(qv, qo + 2)

            def half_body(jh, accs):
                t16 = i * H + jh * 16
                idx16 = nb_v[si, pl.ds(t16, 16)]
                xs = plsc.load_gather(spts_v, [idx16])
                ys = plsc.load_gather(spts_v, [idx16 + N])
                zs = plsc.load_gather(spts_v, [idx16 + 2 * N])
                dx = xs - qxs
                dy = ys - qys
                dz = zs - qzs
                best = jnp.full((16,), 1e30, jnp.float32)
                bestk = jnp.zeros((16,), jnp.int32)
                for k in range(K):
                    ddx = dx - kpx[k]
                    ddy = dy - kpy[k]
                    ddz = dz - kpz[k]
                    d2 = ddx * ddx + ddy * ddy + ddz * ddz
                    m = d2 < best
                    best = jnp.where(m, d2, best)
                    bestk = jnp.where(m, jnp.int32(k), bestk)
                x = jnp.maximum(best, jnp.float32(1e-24))
                xi = plsc.bitcast(x, jnp.int32)
                r = plsc.bitcast(jnp.int32(0x5F3759DF) - (xi >> 1),
                                 jnp.float32)
                for _ in range(3):
                    r = r * (1.5 - 0.5 * x * r * r)
                s = x * r
                infl16 = jnp.maximum(1.0 - s * jnp.float32(1.0 / SIGMA), 0.0)
                kofs16 = bestk * CPG

                accs = list(accs)
                for l in range(16):
                    kofs = kofs16[l]
                    fl = infl16[l]
                    m16 = mod_v[si, pl.ds(i * MODW + kofs, 16)]
                    modi = m16 * fl
                    n = t16 + l
                    for g in range(GROUPS):
                        w = wts_v[pl.ds(kofs * GROUPS + g * CPG, 16)]
                        f = feat_v[sf, n, pl.ds(g * CPG, 16)]
                        accs[g] = accs[g] + f * w * modi
                return tuple(accs)

            accs = lax.fori_loop(
                0, 2, half_body,
                tuple(jnp.zeros((16,), jnp.float32) for _ in range(GROUPS)))
            for g in range(GROUPS):
                out_v[i, pl.ds(g * CPG, 16)] = accs[g]
            return carry_q

        lax.fori_loop(0, QB, q_body, 0)
        pltpu.async_copy(out_v, out_hbm.at[pl.ds(base, QB)], sem_o)
        return carry

    lax.fori_loop(0, nchunks, chunk_body, 0)
    wait_out(nchunks - 1)


def _sc_main(nb_flat, qp_pad, spts_flat, s_feats, wts_flat, kp, mod_flat):
    mesh = plsc.VectorSubcoreMesh(core_axis_name="c", subcore_axis_name="s",
                                  num_cores=NC, num_subcores=NS)
    return pl.kernel(
        _sc_body,
        out_type=jax.ShapeDtypeStruct((M, C), jnp.float32),
        mesh=mesh,
        compiler_params=pltpu.CompilerParams(needs_layout_passes=False),
        scratch_types=[
            pltpu.VMEM((3 * N,), jnp.float32),
            pltpu.VMEM((K * C,), jnp.float32),
            pltpu.VMEM((48,), jnp.float32),
            pltpu.VMEM((3, QB * H), jnp.int32),
            pltpu.VMEM((3, QB * 4), jnp.float32),
            pltpu.VMEM((3, QB * MODW), jnp.float32),
            pltpu.VMEM((2, QB * H, C), jnp.float32),
            pltpu.VMEM((QB, C), jnp.float32),
            pltpu.SemaphoreType.DMA,
            pltpu.SemaphoreType.DMA,
            pltpu.SemaphoreType.DMA,
        ],
    )(nb_flat, qp_pad, spts_flat, s_feats, wts_flat, kp, mod_flat)


def kernel(q_pts, s_pts, s_feats, neighb_inds, weights, W1, b1, W2,
           kernel_points):
    nb_flat = neighb_inds.astype(jnp.int32).reshape(-1)
    qp_flat = jnp.pad(q_pts, ((0, 0), (0, 1))).reshape(-1)
    spts_flat = s_pts.T.reshape(-1)
    wts_flat = weights.reshape(-1)
    kp_flat = jnp.pad(kernel_points, ((0, 16 - K), (0, 0))).T.reshape(-1)
    W2p = jnp.pad(W2, ((0, 0), (0, MODW - K * CPG)))
    mod = _modulations(s_feats, W1, b1, W2p)
    return _sc_main(nb_flat, qp_flat, spts_flat, s_feats, wts_flat,
                    kp_flat, mod.reshape(-1))
